# Initial kernel scaffold; baseline (speedup 1.0000x reference)
import jax
import jax.numpy as jnp
from jax.experimental import pallas as pl


def kernel(predictions, targets_boxes, targets_labels):
    raise NotImplementedError("write your pallas kernel here")


# speedup vs baseline: 524.9123x; 524.9123x over previous
import jax
import jax.numpy as jnp
from jax import lax
from jax.experimental import pallas as pl
from jax.experimental.pallas import tpu as pltpu

_NUM_CLASSES = 5
_H = _W = 13
_S = _H * _W
_A = 3
_ANCHOR_W = (0.05, 0.1, 0.15)
_CH = 2 * _NUM_CLASSES
_M = _A * _CH * _S


def _loss_body(pred_ref, cx_ref, cy_ref, w_ref, h_ref, labels_ref, out_ref):
    pred = pred_ref[...]
    cx = cx_ref[...]
    cy = cy_ref[...]
    w = w_ref[...]
    h = h_ref[...]
    labels = labels_ref[...]
    B = pred.shape[0]
    N = cx.shape[1]

    valid = (cx > 0) & (cx < 1) & (cy > 0) & (cy < 1) & (w > 0) & (h > 0)
    gx = jnp.clip(jnp.floor(cx * _W).astype(jnp.int32), 0, _W - 1)
    gy = jnp.clip(jnp.floor(cy * _H).astype(jnp.int32), 0, _H - 1)

    ious = []
    for aw in _ANCHOR_W:
        inter = jnp.minimum(w, aw) * jnp.minimum(h, aw)
        ious.append(inter / (w * h + aw * aw - inter))
    best = jnp.where(ious[1] > ious[0], 1, 0).astype(jnp.int32)
    best = jnp.where(ious[2] > jnp.maximum(ious[0], ious[1]), 2, best)
    awb = jnp.where(best == 0, _ANCHOR_W[0],
                    jnp.where(best == 1, _ANCHOR_W[1], _ANCHOR_W[2]))

    tx = cx * _W - gx.astype(jnp.float32)
    ty = cy * _H - gy.astype(jnp.float32)
    tw = jnp.log(w / awb + 1e-16)
    th = jnp.log(h / awb + 1e-16)
    boxkey = best * _S + gy * _W + gx
    label_ok = (labels >= 0) & (labels < _NUM_CLASSES)

    m_i = lax.broadcasted_iota(jnp.int32, (B, _M), 1)
    ch_j = (m_i // _S) % _CH
    cellkey = (m_i // (_CH * _S)) * _S + m_i % _S
    is_obj = ch_j == 0
    is_bbox = (ch_j >= 1) & (ch_j <= 4)
    cls_id = ch_j - _NUM_CLASSES

    cellmask = jnp.zeros((B, _M), jnp.float32)
    tgt = jnp.zeros((B, _M), jnp.float32)
    for i in range(N):
        b2 = lambda x: x[:, i][:, None]
        match = b2(valid) & (cellkey == b2(boxkey))
        vb = jnp.where(ch_j == 1, b2(tx),
                       jnp.where(ch_j == 2, b2(ty),
                                 jnp.where(ch_j == 3, b2(tw), b2(th))))
        tgt = jnp.where(match & is_bbox, vb, tgt)
        set1 = is_obj | ((cls_id == b2(labels)) & b2(label_ok))
        tgt = jnp.where(match & set1, 1.0, tgt)
        cellmask = jnp.where(match, 1.0, cellmask)

    cnt = jnp.sum(jnp.where(is_obj, cellmask, 0.0), axis=1)
    pw = (float(_A * _S) - cnt) / (cnt + 1e-16)

    bce0 = jnp.maximum(pred, 0.0) + jnp.log1p(jnp.exp(-jnp.abs(pred)))
    bce = bce0 - pred * tgt
    ol = jnp.where(cellmask > 0, bce * pw[:, None], bce)
    obj_b = jnp.sum(jnp.where(is_obj, ol, 0.0), axis=1) / float(_A * _S)

    mse = (pred - tgt) ** 2
    bbox_sum = jnp.sum(jnp.where(is_bbox, mse * cellmask, 0.0), axis=1)
    cls_sum = jnp.sum(jnp.where(cls_id >= 0, bce * cellmask, 0.0), axis=1)

    has = cnt > 0
    bbox_b = jnp.where(has, bbox_sum / (4.0 * cnt), 0.0)
    cls_b = jnp.where(has, cls_sum / (float(_NUM_CLASSES) * cnt), 0.0)
    total_obj = jnp.sum(obj_b) / B
    any_has = jnp.any(has)
    total_bbox = jnp.where(any_has, jnp.sum(bbox_b) / B, 0.0)
    total_cls = jnp.where(any_has, jnp.sum(cls_b) / B, 0.0)
    out_ref[0, 0] = 2.0 * total_obj + 5.0 * total_bbox + 2.0 * total_cls


def kernel(predictions, targets_boxes, targets_labels):
    B = predictions.shape[0]
    pred = predictions.reshape(B, _M)
    cx = targets_boxes[:, :, 0]
    cy = targets_boxes[:, :, 1]
    w = targets_boxes[:, :, 2]
    h = targets_boxes[:, :, 3]
    out = pl.pallas_call(
        _loss_body,
        out_shape=jax.ShapeDtypeStruct((1, 1), jnp.float32),
        out_specs=pl.BlockSpec(memory_space=pltpu.SMEM),
    )(pred, cx, cy, w, h, targets_labels)
    return out[0, 0]
